# baseline (speedup 1.0000x reference)
import functools

import jax
import jax.numpy as jnp
from jax import lax
from jax.experimental import pallas as pl
from jax.experimental.pallas import tpu as pltpu
from jax.experimental.pallas import tpu_sc as plsc

_C = 64
_L = 16
_NSC = 2
_NTL = 16
_T1 = 1280
_T2 = 896
_HB = 128
_NT = _HB // _L

_RB = 512


def _sc_body(logT_h, nTinv_h, den_h, yT_h,
             a1_h, b1_h, g1_h, r11_h, p11_h, p21_h,
             a2_h, b2_h, g2_h, r12_h, r22_h, p12_h, p22_h,
             out_h,
             logT_v, nTinv_v, den_v, a16_b, b16_b, g16_b,
             r1_b, r2_b, p1_b, p2_b, r1a_b, r2a_b,
             rs1_b, rs2_b, sp1_b, sp2_b,
             y1_b, y2_b, term_b, nterm_b, psem, gsem, ssem, dy_sp):
    c = lax.axis_index("c")
    s = lax.axis_index("s")
    nrow = dy_sp.shape[0] // _NTL
    coff = c * yT_h.shape[0] // _NSC

    pltpu.sync_copy(logT_h.at[pl.ds(c * _HB, _HB)], logT_v)
    pltpu.sync_copy(nTinv_h.at[pl.ds(c * _HB, _HB)], nTinv_v)
    pltpu.sync_copy(den_h.at[pl.ds(c * _HB, _HB)], den_v)

    zero = jnp.zeros((_L,), jnp.float32)
    tz = term_b.at[0]

    def zrow(i, carry):
        for t in range(_NT):
            tz[i, pl.ds(t * _L, _L)] = zero
        return carry

    lax.fori_loop(0, nrow, zrow, 0)
    pltpu.sync_copy(tz.at[pl.ds(0, nrow)], dy_sp.at[pl.ds(s * nrow, nrow)])
    plsc.subcore_barrier()

    meds = [(logT_v[pl.ds(t * _L, _L)],
             nTinv_v[pl.ds(t * _L, _L)],
             den_v[pl.ds(t * _L, _L)]) for t in range(_NT)]

    def run_phase(second, Tlen, a_h, b_h, g_h, r1_h, r2_h, p1_h, p2_h):
        n = Tlen // _C

        def issue_params(k, p):
            base = s * Tlen + k * _C
            pltpu.async_copy(a_h.at[pl.ds(base * _L, _C * _L)], a16_b.at[p], psem.at[p])
            pltpu.async_copy(b_h.at[pl.ds(base * _L, _C * _L)], b16_b.at[p], psem.at[p])
            pltpu.async_copy(g_h.at[pl.ds(base * _L, _C * _L)], g16_b.at[p], psem.at[p])
            pltpu.async_copy(r1_h.at[pl.ds(base, _C)], r1_b.at[p], psem.at[p])
            if second:
                pltpu.async_copy(r2_h.at[pl.ds(base, _C)], r2_b.at[p], psem.at[p])
            pltpu.async_copy(p1_h.at[pl.ds(base, _C)], p1_b.at[p], psem.at[p])
            pltpu.async_copy(p2_h.at[pl.ds(base, _C)], p2_b.at[p], psem.at[p])

        def prep(p):
            pltpu.make_async_copy(a_h.at[pl.ds(0, _C * _L)], a16_b.at[p], psem.at[p]).wait()
            pltpu.make_async_copy(b_h.at[pl.ds(0, _C * _L)], b16_b.at[p], psem.at[p]).wait()
            pltpu.make_async_copy(g_h.at[pl.ds(0, _C * _L)], g16_b.at[p], psem.at[p]).wait()
            pltpu.make_async_copy(r1_h.at[pl.ds(0, _C)], r1_b.at[p], psem.at[p]).wait()
            if second:
                pltpu.make_async_copy(r2_h.at[pl.ds(0, _C)], r2_b.at[p], psem.at[p]).wait()
            pltpu.make_async_copy(p1_h.at[pl.ds(0, _C)], p1_b.at[p], psem.at[p]).wait()
            pltpu.make_async_copy(p2_h.at[pl.ds(0, _C)], p2_b.at[p], psem.at[p]).wait()
            for t in range(_C // _L):
                r1a_b[p, pl.ds(t * _L, _L)] = r1_b[p, pl.ds(t * _L, _L)] + coff
                if second:
                    r2a_b[p, pl.ds(t * _L, _L)] = r2_b[p, pl.ds(t * _L, _L)] + coff
            pltpu.async_copy(yT_h.at[r1a_b.at[p]], y1_b.at[p], gsem.at[p])
            if second:
                pltpu.async_copy(yT_h.at[r2a_b.at[p]], y2_b.at[p], gsem.at[p])

        def wait_gathers(p):
            pltpu.make_async_copy(yT_h.at[r1a_b.at[p]], y1_b.at[p], gsem.at[p]).wait()
            if second:
                pltpu.make_async_copy(yT_h.at[r2a_b.at[p]], y2_b.at[p], gsem.at[p]).wait()

        def compute(p):
            a16 = a16_b.at[p]
            b16 = b16_b.at[p]
            g16 = g16_b.at[p]
            y1v = y1_b.at[p]
            y2v = y2_b.at[p]
            termv = term_b.at[p]
            ntermv = nterm_b.at[p]

            def jbody(j, inner):
                al = a16[pl.ds(j * _L, _L)]
                be = b16[pl.ds(j * _L, _L)]
                ga = g16[pl.ds(j * _L, _L)]
                for t in range(_NT):
                    logT, nTinv, den = meds[t]
                    rate = al * jnp.exp(be * logT + ga * nTinv)
                    if second:
                        rate = rate * den
                        term = rate * y1v[j, pl.ds(t * _L, _L)] * y2v[j, pl.ds(t * _L, _L)]
                    else:
                        term = rate * y1v[j, pl.ds(t * _L, _L)]
                    termv[j, pl.ds(t * _L, _L)] = term
                    ntermv[j, pl.ds(t * _L, _L)] = -term
                return inner

            lax.fori_loop(0, _C, jbody, 0)

        def issue_scatters(p):
            for t in range(_C // _L):
                rs1_b[p, pl.ds(t * _L, _L)] = r1_b[p, pl.ds(t * _L, _L)]
                if second:
                    rs2_b[p, pl.ds(t * _L, _L)] = r2_b[p, pl.ds(t * _L, _L)]
                sp1_b[p, pl.ds(t * _L, _L)] = p1_b[p, pl.ds(t * _L, _L)]
                sp2_b[p, pl.ds(t * _L, _L)] = p2_b[p, pl.ds(t * _L, _L)]
            pltpu.async_copy(nterm_b.at[p], dy_sp.at[rs1_b.at[p]], ssem.at[p], add=True)
            if second:
                pltpu.async_copy(nterm_b.at[p], dy_sp.at[rs2_b.at[p]], ssem.at[p], add=True)
            pltpu.async_copy(term_b.at[p], dy_sp.at[sp1_b.at[p]], ssem.at[p], add=True)
            pltpu.async_copy(term_b.at[p], dy_sp.at[sp2_b.at[p]], ssem.at[p], add=True)

        def wait_scatters(p):
            pltpu.make_async_copy(nterm_b.at[p], dy_sp.at[rs1_b.at[p]], ssem.at[p]).wait()
            if second:
                pltpu.make_async_copy(nterm_b.at[p], dy_sp.at[rs2_b.at[p]], ssem.at[p]).wait()
            pltpu.make_async_copy(term_b.at[p], dy_sp.at[sp1_b.at[p]], ssem.at[p]).wait()
            pltpu.make_async_copy(term_b.at[p], dy_sp.at[sp2_b.at[p]], ssem.at[p]).wait()

        issue_params(0, 0)
        prep(0)
        issue_params(1, 1)

        prep(1)
        wait_gathers(0)
        compute(0)
        issue_scatters(0)
        issue_params(2, 0)
        prep(0)
        wait_gathers(1)
        compute(1)
        issue_scatters(1)
        issue_params(3, 1)

        def body(cp, carry):
            k0 = 2 * cp
            prep(1)
            wait_gathers(0)
            wait_scatters(0)
            compute(0)
            issue_scatters(0)
            issue_params(k0 + 2, 0)
            prep(0)
            wait_gathers(1)
            wait_scatters(1)
            compute(1)
            issue_scatters(1)
            issue_params(k0 + 3, 1)
            return carry

        lax.fori_loop(1, n // 2 - 1, body, 0)

        prep(1)
        wait_gathers(0)
        wait_scatters(0)
        compute(0)
        issue_scatters(0)
        wait_gathers(1)
        wait_scatters(1)
        compute(1)
        issue_scatters(1)
        wait_scatters(0)
        wait_scatters(1)

    run_phase(False, _T1, a1_h, b1_h, g1_h, r11_h, None, p11_h, p21_h)
    run_phase(True, _T2, a2_h, b2_h, g2_h, r12_h, r22_h, p12_h, p22_h)

    plsc.subcore_barrier()
    pltpu.sync_copy(dy_sp.at[pl.ds(s * nrow, nrow)],
                    out_h.at[c, pl.ds(s * nrow, nrow)])


def _tc_body(t_ref, wT_ref, bT_ref, wd_ref, bd_ref, y_ref,
             al_ref, be_ref, ga_ref, r1_ref, r2_ref, p1_ref, p2_ref,
             out_ref, *, n_spec):
    i = pl.program_id(0)

    t = t_ref[...]
    T = jnp.exp(wT_ref[0] * t + bT_ref[0]) + 10.0
    den = jnp.exp(wd_ref[0] * t + bd_ref[0])
    logT = jnp.log(T / 300.0)
    nTinv = -1.0 / T

    al = al_ref[0]
    be = be_ref[0]
    ga = ga_ref[0]
    rates = den * al * jnp.exp(be * logT + ga * nTinv)

    r1 = r1_ref[0]
    r2 = r2_ref[0]
    p1 = p1_ref[0]
    p2 = p2_ref[0]

    spec = lax.broadcasted_iota(jnp.int32, (n_spec, _RB), 0)
    G1 = (spec == r1).astype(jnp.float32)
    G2 = (spec == r2).astype(jnp.float32)
    P1 = (spec == p1).astype(jnp.float32)
    P2 = (spec == p2).astype(jnp.float32)

    y = y_ref[...]
    Y1 = jnp.dot(y, G1, preferred_element_type=jnp.float32)
    Y2 = jnp.dot(y, G2, preferred_element_type=jnp.float32)

    term = rates * Y1 * Y2

    S = P1 + P2 - G1 - G2
    contrib = lax.dot_general(term, S, (((1,), (1,)), ((), ())),
                              preferred_element_type=jnp.float32)

    @pl.when(i == 0)
    def _():
        out_ref[...] = jnp.zeros_like(out_ref)

    out_ref[...] += contrib


def _bcast16(x, n_pad):
    x = jnp.pad(x.astype(jnp.float32), (0, n_pad - x.shape[0]))
    return jnp.repeat(x[:, None], _L, axis=1).reshape(-1)


def _padi(x, n_pad):
    return jnp.pad(x.astype(jnp.int32), (0, n_pad - x.shape[0]))


def kernel(t_in, y_in, alpha_1st, beta_1st, gamma_1st, alpha_2nd, beta_2nd,
           gamma_2nd, w_T, b_T, w_d, b_d, inds_r1_1st, inds_p1_1st,
           inds_p2_1st, inds_r1_2nd, inds_r2_2nd, inds_p1_2nd, inds_p2_2nd):
    B = t_in.shape[0]
    nspec = y_in.shape[1]
    f32 = jnp.float32
    i32 = jnp.int32
    n1p = _T1 * _NTL
    n2p = _T2 * _NTL

    T_gas = jnp.exp(w_T * t_in + b_T) + 10.0
    den = jnp.exp(w_d * t_in + b_d).astype(f32)
    logT = jnp.log(T_gas / 300.0).astype(f32)
    nTinv = (-1.0 / T_gas).astype(f32)

    y = y_in.astype(f32)
    yT = jnp.concatenate([y[:_HB].T, y[_HB:].T], axis=0)

    scratch = [
        pltpu.VMEM((_HB,), f32),
        pltpu.VMEM((_HB,), f32),
        pltpu.VMEM((_HB,), f32),
        pltpu.VMEM((2, _C * _L), f32),
        pltpu.VMEM((2, _C * _L), f32),
        pltpu.VMEM((2, _C * _L), f32),
        pltpu.VMEM((2, _C), i32),
        pltpu.VMEM((2, _C), i32),
        pltpu.VMEM((2, _C), i32),
        pltpu.VMEM((2, _C), i32),
        pltpu.VMEM((2, _C), i32),
        pltpu.VMEM((2, _C), i32),
        pltpu.VMEM((2, _C), i32),
        pltpu.VMEM((2, _C), i32),
        pltpu.VMEM((2, _C), i32),
        pltpu.VMEM((2, _C), i32),
        pltpu.VMEM((2, _C, _HB), f32),
        pltpu.VMEM((2, _C, _HB), f32),
        pltpu.VMEM((2, _C, _HB), f32),
        pltpu.VMEM((2, _C, _HB), f32),
        pltpu.SemaphoreType.DMA((2,)),
        pltpu.SemaphoreType.DMA((2,)),
        pltpu.SemaphoreType.DMA((2,)),
        pltpu.VMEM_SHARED((nspec, _HB), f32),
    ]

    sc_call = pl.kernel(
        _sc_body,
        out_type=jax.ShapeDtypeStruct((_NSC, nspec, _HB), f32),
        mesh=plsc.VectorSubcoreMesh(core_axis_name="c", subcore_axis_name="s"),
        scratch_types=scratch,
    )
    sc_out = sc_call(
        logT, nTinv, den, yT,
        _bcast16(alpha_1st, n1p), _bcast16(beta_1st, n1p),
        _bcast16(gamma_1st, n1p),
        _padi(inds_r1_1st, n1p), _padi(inds_p1_1st, n1p),
        _padi(inds_p2_1st, n1p),
        _bcast16(alpha_2nd[:n2p], n2p), _bcast16(beta_2nd[:n2p], n2p),
        _bcast16(gamma_2nd[:n2p], n2p),
        _padi(inds_r1_2nd[:n2p], n2p), _padi(inds_r2_2nd[:n2p], n2p),
        _padi(inds_p1_2nd[:n2p], n2p), _padi(inds_p2_2nd[:n2p], n2p),
    )
    dy_sc = jnp.concatenate([sc_out[0], sc_out[1]], axis=1).T

    R2n = alpha_2nd.shape[0] - n2p
    nb = -(-R2n // _RB)
    pad = nb * _RB - R2n

    def padded(x, fill):
        x = jnp.pad(x[n2p:], (0, pad), constant_values=fill)
        return x.reshape(nb, 1, _RB)

    al = padded(alpha_2nd.astype(f32), 0.0)
    be = padded(beta_2nd.astype(f32), 0.0)
    ga = padded(gamma_2nd.astype(f32), 0.0)
    r1 = padded(inds_r1_2nd.astype(i32), 0)
    r2 = padded(inds_r2_2nd.astype(i32), 0)
    p1 = padded(inds_p1_2nd.astype(i32), 0)
    p2 = padded(inds_p2_2nd.astype(i32), 0)

    t2 = t_in.reshape(B, 1)

    full2d = lambda shape: pl.BlockSpec(shape, lambda i: (0, 0))
    par3d = pl.BlockSpec((1, 1, _RB), lambda i: (i, 0, 0))
    smem = pl.BlockSpec(memory_space=pltpu.SMEM)

    dy_tc = pl.pallas_call(
        functools.partial(_tc_body, n_spec=nspec),
        grid=(nb,),
        in_specs=[full2d((B, 1)), smem, smem, smem, smem, full2d((B, nspec)),
                  par3d, par3d, par3d, par3d, par3d, par3d, par3d],
        out_specs=pl.BlockSpec((B, nspec), lambda i: (0, 0)),
        out_shape=jax.ShapeDtypeStruct((B, nspec), f32),
    )(t2, w_T, b_T, w_d, b_d, y, al, be, ga, r1, r2, p1, p2)

    return dy_sc + dy_tc


# speedup vs baseline: 1.1342x; 1.1342x over previous
import functools

import jax
import jax.numpy as jnp
from jax import lax
from jax.experimental import pallas as pl
from jax.experimental.pallas import tpu as pltpu
from jax.experimental.pallas import tpu_sc as plsc

_C = 64
_L = 16
_NSC = 2
_NTL = 16
_T1 = 1280
_T2 = 640
_HB = 128
_NT = _HB // _L

_RB = 1024


def _sc_body(logT_h, nTinv_h, den_h, yT_h,
             a1_h, b1_h, g1_h, r11_h, p11_h, p21_h,
             a2_h, b2_h, g2_h, r12_h, r22_h, p12_h, p22_h,
             out_h,
             logT_v, nTinv_v, den_v, a16_b, b16_b, g16_b,
             r1_b, r2_b, p1_b, p2_b, r1a_b, r2a_b,
             rs1_b, rs2_b, sp1_b, sp2_b,
             y1_b, y2_b, term_b, nterm_b, psem, gsem, ssem, dy_sp):
    c = lax.axis_index("c")
    s = lax.axis_index("s")
    nrow = dy_sp.shape[0] // _NTL
    coff = c * yT_h.shape[0] // _NSC

    pltpu.sync_copy(logT_h.at[pl.ds(c * _HB, _HB)], logT_v)
    pltpu.sync_copy(nTinv_h.at[pl.ds(c * _HB, _HB)], nTinv_v)
    pltpu.sync_copy(den_h.at[pl.ds(c * _HB, _HB)], den_v)

    zero = jnp.zeros((_L,), jnp.float32)
    tz = term_b.at[0]

    def zrow(i, carry):
        for t in range(_NT):
            tz[i, pl.ds(t * _L, _L)] = zero
        return carry

    lax.fori_loop(0, nrow, zrow, 0)
    pltpu.sync_copy(tz.at[pl.ds(0, nrow)], dy_sp.at[pl.ds(s * nrow, nrow)])
    plsc.subcore_barrier()

    meds = [(logT_v[pl.ds(t * _L, _L)],
             nTinv_v[pl.ds(t * _L, _L)],
             den_v[pl.ds(t * _L, _L)]) for t in range(_NT)]

    def run_phase(second, Tlen, a_h, b_h, g_h, r1_h, r2_h, p1_h, p2_h):
        n = Tlen // _C

        def issue_params(k, p):
            base = s * Tlen + k * _C
            pltpu.async_copy(a_h.at[pl.ds(base * _L, _C * _L)], a16_b.at[p], psem.at[p])
            pltpu.async_copy(b_h.at[pl.ds(base * _L, _C * _L)], b16_b.at[p], psem.at[p])
            pltpu.async_copy(g_h.at[pl.ds(base * _L, _C * _L)], g16_b.at[p], psem.at[p])
            pltpu.async_copy(r1_h.at[pl.ds(base, _C)], r1_b.at[p], psem.at[p])
            if second:
                pltpu.async_copy(r2_h.at[pl.ds(base, _C)], r2_b.at[p], psem.at[p])
            pltpu.async_copy(p1_h.at[pl.ds(base, _C)], p1_b.at[p], psem.at[p])
            pltpu.async_copy(p2_h.at[pl.ds(base, _C)], p2_b.at[p], psem.at[p])

        def prep(p):
            pltpu.make_async_copy(a_h.at[pl.ds(0, _C * _L)], a16_b.at[p], psem.at[p]).wait()
            pltpu.make_async_copy(b_h.at[pl.ds(0, _C * _L)], b16_b.at[p], psem.at[p]).wait()
            pltpu.make_async_copy(g_h.at[pl.ds(0, _C * _L)], g16_b.at[p], psem.at[p]).wait()
            pltpu.make_async_copy(r1_h.at[pl.ds(0, _C)], r1_b.at[p], psem.at[p]).wait()
            if second:
                pltpu.make_async_copy(r2_h.at[pl.ds(0, _C)], r2_b.at[p], psem.at[p]).wait()
            pltpu.make_async_copy(p1_h.at[pl.ds(0, _C)], p1_b.at[p], psem.at[p]).wait()
            pltpu.make_async_copy(p2_h.at[pl.ds(0, _C)], p2_b.at[p], psem.at[p]).wait()
            for t in range(_C // _L):
                r1a_b[p, pl.ds(t * _L, _L)] = r1_b[p, pl.ds(t * _L, _L)] + coff
                if second:
                    r2a_b[p, pl.ds(t * _L, _L)] = r2_b[p, pl.ds(t * _L, _L)] + coff
            pltpu.async_copy(yT_h.at[r1a_b.at[p]], y1_b.at[p], gsem.at[p])
            if second:
                pltpu.async_copy(yT_h.at[r2a_b.at[p]], y2_b.at[p], gsem.at[p])

        def wait_gathers(p):
            pltpu.make_async_copy(yT_h.at[r1a_b.at[p]], y1_b.at[p], gsem.at[p]).wait()
            if second:
                pltpu.make_async_copy(yT_h.at[r2a_b.at[p]], y2_b.at[p], gsem.at[p]).wait()

        def compute(p):
            a16 = a16_b.at[p]
            b16 = b16_b.at[p]
            g16 = g16_b.at[p]
            y1v = y1_b.at[p]
            y2v = y2_b.at[p]
            termv = term_b.at[p]
            ntermv = nterm_b.at[p]

            def jbody(j, inner):
                al = a16[pl.ds(j * _L, _L)]
                be = b16[pl.ds(j * _L, _L)]
                ga = g16[pl.ds(j * _L, _L)]
                for t in range(_NT):
                    logT, nTinv, den = meds[t]
                    rate = al * jnp.exp(be * logT + ga * nTinv)
                    if second:
                        rate = rate * den
                        term = rate * y1v[j, pl.ds(t * _L, _L)] * y2v[j, pl.ds(t * _L, _L)]
                    else:
                        term = rate * y1v[j, pl.ds(t * _L, _L)]
                    termv[j, pl.ds(t * _L, _L)] = term
                    ntermv[j, pl.ds(t * _L, _L)] = -term
                return inner

            lax.fori_loop(0, _C, jbody, 0)

        def issue_scatters(p):
            for t in range(_C // _L):
                rs1_b[p, pl.ds(t * _L, _L)] = r1_b[p, pl.ds(t * _L, _L)]
                if second:
                    rs2_b[p, pl.ds(t * _L, _L)] = r2_b[p, pl.ds(t * _L, _L)]
                sp1_b[p, pl.ds(t * _L, _L)] = p1_b[p, pl.ds(t * _L, _L)]
                sp2_b[p, pl.ds(t * _L, _L)] = p2_b[p, pl.ds(t * _L, _L)]
            pltpu.async_copy(nterm_b.at[p], dy_sp.at[rs1_b.at[p]], ssem.at[p], add=True)
            if second:
                pltpu.async_copy(nterm_b.at[p], dy_sp.at[rs2_b.at[p]], ssem.at[p], add=True)
            pltpu.async_copy(term_b.at[p], dy_sp.at[sp1_b.at[p]], ssem.at[p], add=True)
            pltpu.async_copy(term_b.at[p], dy_sp.at[sp2_b.at[p]], ssem.at[p], add=True)

        def wait_scatters(p):
            pltpu.make_async_copy(nterm_b.at[p], dy_sp.at[rs1_b.at[p]], ssem.at[p]).wait()
            if second:
                pltpu.make_async_copy(nterm_b.at[p], dy_sp.at[rs2_b.at[p]], ssem.at[p]).wait()
            pltpu.make_async_copy(term_b.at[p], dy_sp.at[sp1_b.at[p]], ssem.at[p]).wait()
            pltpu.make_async_copy(term_b.at[p], dy_sp.at[sp2_b.at[p]], ssem.at[p]).wait()

        issue_params(0, 0)
        prep(0)
        issue_params(1, 1)

        prep(1)
        wait_gathers(0)
        compute(0)
        issue_scatters(0)
        issue_params(2, 0)
        prep(0)
        wait_gathers(1)
        compute(1)
        issue_scatters(1)
        issue_params(3, 1)

        def body(cp, carry):
            k0 = 2 * cp
            prep(1)
            wait_gathers(0)
            wait_scatters(0)
            compute(0)
            issue_scatters(0)
            issue_params(k0 + 2, 0)
            prep(0)
            wait_gathers(1)
            wait_scatters(1)
            compute(1)
            issue_scatters(1)
            issue_params(k0 + 3, 1)
            return carry

        lax.fori_loop(1, n // 2 - 1, body, 0)

        prep(1)
        wait_gathers(0)
        wait_scatters(0)
        compute(0)
        issue_scatters(0)
        wait_gathers(1)
        wait_scatters(1)
        compute(1)
        issue_scatters(1)
        wait_scatters(0)
        wait_scatters(1)

    run_phase(False, _T1, a1_h, b1_h, g1_h, r11_h, None, p11_h, p21_h)
    run_phase(True, _T2, a2_h, b2_h, g2_h, r12_h, r22_h, p12_h, p22_h)

    plsc.subcore_barrier()
    pltpu.sync_copy(dy_sp.at[pl.ds(s * nrow, nrow)],
                    out_h.at[c, pl.ds(s * nrow, nrow)])


def _tc_body(t_ref, wT_ref, bT_ref, wd_ref, bd_ref, y_ref,
             al_ref, be_ref, ga_ref, r1_ref, r2_ref, p1_ref, p2_ref,
             out_ref, *, n_spec):
    i = pl.program_id(0)

    t = t_ref[...]
    T = jnp.exp(wT_ref[0] * t + bT_ref[0]) + 10.0
    den = jnp.exp(wd_ref[0] * t + bd_ref[0])
    logT = jnp.log(T / 300.0)
    nTinv = -1.0 / T

    al = al_ref[0]
    be = be_ref[0]
    ga = ga_ref[0]
    rates = den * al * jnp.exp(be * logT + ga * nTinv)

    r1 = r1_ref[0]
    r2 = r2_ref[0]
    p1 = p1_ref[0]
    p2 = p2_ref[0]

    spec = lax.broadcasted_iota(jnp.int32, (n_spec, _RB), 0)
    G1 = (spec == r1).astype(jnp.float32)
    G2 = (spec == r2).astype(jnp.float32)
    P1 = (spec == p1).astype(jnp.float32)
    P2 = (spec == p2).astype(jnp.float32)

    y = y_ref[...]
    Y1 = jnp.dot(y, G1, preferred_element_type=jnp.float32)
    Y2 = jnp.dot(y, G2, preferred_element_type=jnp.float32)

    term = rates * Y1 * Y2

    S = P1 + P2 - G1 - G2
    contrib = lax.dot_general(term, S, (((1,), (1,)), ((), ())),
                              preferred_element_type=jnp.float32)

    @pl.when(i == 0)
    def _():
        out_ref[...] = jnp.zeros_like(out_ref)

    out_ref[...] += contrib


def _bcast16(x, n_pad):
    x = jnp.pad(x.astype(jnp.float32), (0, n_pad - x.shape[0]))
    return jnp.repeat(x[:, None], _L, axis=1).reshape(-1)


def _padi(x, n_pad):
    return jnp.pad(x.astype(jnp.int32), (0, n_pad - x.shape[0]))


def kernel(t_in, y_in, alpha_1st, beta_1st, gamma_1st, alpha_2nd, beta_2nd,
           gamma_2nd, w_T, b_T, w_d, b_d, inds_r1_1st, inds_p1_1st,
           inds_p2_1st, inds_r1_2nd, inds_r2_2nd, inds_p1_2nd, inds_p2_2nd):
    B = t_in.shape[0]
    nspec = y_in.shape[1]
    f32 = jnp.float32
    i32 = jnp.int32
    n1p = _T1 * _NTL
    n2p = _T2 * _NTL

    T_gas = jnp.exp(w_T * t_in + b_T) + 10.0
    den = jnp.exp(w_d * t_in + b_d).astype(f32)
    logT = jnp.log(T_gas / 300.0).astype(f32)
    nTinv = (-1.0 / T_gas).astype(f32)

    y = y_in.astype(f32)
    yT = jnp.concatenate([y[:_HB].T, y[_HB:].T], axis=0)

    scratch = [
        pltpu.VMEM((_HB,), f32),
        pltpu.VMEM((_HB,), f32),
        pltpu.VMEM((_HB,), f32),
        pltpu.VMEM((2, _C * _L), f32),
        pltpu.VMEM((2, _C * _L), f32),
        pltpu.VMEM((2, _C * _L), f32),
        pltpu.VMEM((2, _C), i32),
        pltpu.VMEM((2, _C), i32),
        pltpu.VMEM((2, _C), i32),
        pltpu.VMEM((2, _C), i32),
        pltpu.VMEM((2, _C), i32),
        pltpu.VMEM((2, _C), i32),
        pltpu.VMEM((2, _C), i32),
        pltpu.VMEM((2, _C), i32),
        pltpu.VMEM((2, _C), i32),
        pltpu.VMEM((2, _C), i32),
        pltpu.VMEM((2, _C, _HB), f32),
        pltpu.VMEM((2, _C, _HB), f32),
        pltpu.VMEM((2, _C, _HB), f32),
        pltpu.VMEM((2, _C, _HB), f32),
        pltpu.SemaphoreType.DMA((2,)),
        pltpu.SemaphoreType.DMA((2,)),
        pltpu.SemaphoreType.DMA((2,)),
        pltpu.VMEM_SHARED((nspec, _HB), f32),
    ]

    sc_call = pl.kernel(
        _sc_body,
        out_type=jax.ShapeDtypeStruct((_NSC, nspec, _HB), f32),
        mesh=plsc.VectorSubcoreMesh(core_axis_name="c", subcore_axis_name="s"),
        scratch_types=scratch,
    )
    sc_out = sc_call(
        logT, nTinv, den, yT,
        _bcast16(alpha_1st, n1p), _bcast16(beta_1st, n1p),
        _bcast16(gamma_1st, n1p),
        _padi(inds_r1_1st, n1p), _padi(inds_p1_1st, n1p),
        _padi(inds_p2_1st, n1p),
        _bcast16(alpha_2nd[:n2p], n2p), _bcast16(beta_2nd[:n2p], n2p),
        _bcast16(gamma_2nd[:n2p], n2p),
        _padi(inds_r1_2nd[:n2p], n2p), _padi(inds_r2_2nd[:n2p], n2p),
        _padi(inds_p1_2nd[:n2p], n2p), _padi(inds_p2_2nd[:n2p], n2p),
    )
    dy_sc = jnp.concatenate([sc_out[0], sc_out[1]], axis=1).T

    R2n = alpha_2nd.shape[0] - n2p
    nb = -(-R2n // _RB)
    pad = nb * _RB - R2n

    def padded(x, fill):
        x = jnp.pad(x[n2p:], (0, pad), constant_values=fill)
        return x.reshape(nb, 1, _RB)

    al = padded(alpha_2nd.astype(f32), 0.0)
    be = padded(beta_2nd.astype(f32), 0.0)
    ga = padded(gamma_2nd.astype(f32), 0.0)
    r1 = padded(inds_r1_2nd.astype(i32), 0)
    r2 = padded(inds_r2_2nd.astype(i32), 0)
    p1 = padded(inds_p1_2nd.astype(i32), 0)
    p2 = padded(inds_p2_2nd.astype(i32), 0)

    t2 = t_in.reshape(B, 1)

    full2d = lambda shape: pl.BlockSpec(shape, lambda i: (0, 0))
    par3d = pl.BlockSpec((1, 1, _RB), lambda i: (i, 0, 0))
    smem = pl.BlockSpec(memory_space=pltpu.SMEM)

    dy_tc = pl.pallas_call(
        functools.partial(_tc_body, n_spec=nspec),
        grid=(nb,),
        in_specs=[full2d((B, 1)), smem, smem, smem, smem, full2d((B, nspec)),
                  par3d, par3d, par3d, par3d, par3d, par3d, par3d],
        out_specs=pl.BlockSpec((B, nspec), lambda i: (0, 0)),
        out_shape=jax.ShapeDtypeStruct((B, nspec), f32),
    )(t2, w_T, b_T, w_d, b_d, y, al, be, ga, r1, r2, p1, p2)

    return dy_sc + dy_tc


# speedup vs baseline: 1.1472x; 1.0114x over previous
import functools

import jax
import jax.numpy as jnp
from jax import lax
from jax.experimental import pallas as pl
from jax.experimental.pallas import tpu as pltpu
from jax.experimental.pallas import tpu_sc as plsc

_C = 64
_L = 16
_NSC = 2
_NTL = 16
_T1 = 1280
_T2 = 640
_HB = 128
_NT = _HB // _L

_RB = 2048


def _sc_body(logT_h, nTinv_h, den_h, yT_h,
             a1_h, b1_h, g1_h, r11_h, p11_h, p21_h,
             a2_h, b2_h, g2_h, r12_h, r22_h, p12_h, p22_h,
             out_h,
             logT_v, nTinv_v, den_v, a16_b, b16_b, g16_b,
             r1_b, r2_b, p1_b, p2_b, r1a_b, r2a_b,
             rs1_b, rs2_b, sp1_b, sp2_b,
             y1_b, y2_b, term_b, nterm_b, psem, gsem, ssem, dy_sp):
    c = lax.axis_index("c")
    s = lax.axis_index("s")
    nrow = dy_sp.shape[0] // _NTL
    coff = c * yT_h.shape[0] // _NSC

    pltpu.sync_copy(logT_h.at[pl.ds(c * _HB, _HB)], logT_v)
    pltpu.sync_copy(nTinv_h.at[pl.ds(c * _HB, _HB)], nTinv_v)
    pltpu.sync_copy(den_h.at[pl.ds(c * _HB, _HB)], den_v)

    zero = jnp.zeros((_L,), jnp.float32)
    tz = term_b.at[0]

    def zrow(i, carry):
        for t in range(_NT):
            tz[i, pl.ds(t * _L, _L)] = zero
        return carry

    lax.fori_loop(0, nrow, zrow, 0)
    pltpu.sync_copy(tz.at[pl.ds(0, nrow)], dy_sp.at[pl.ds(s * nrow, nrow)])
    plsc.subcore_barrier()

    meds = [(logT_v[pl.ds(t * _L, _L)],
             nTinv_v[pl.ds(t * _L, _L)],
             den_v[pl.ds(t * _L, _L)]) for t in range(_NT)]

    def run_phase(second, Tlen, a_h, b_h, g_h, r1_h, r2_h, p1_h, p2_h):
        n = Tlen // _C

        def issue_params(k, p):
            base = s * Tlen + k * _C
            pltpu.async_copy(a_h.at[pl.ds(base * _L, _C * _L)], a16_b.at[p], psem.at[p])
            pltpu.async_copy(b_h.at[pl.ds(base * _L, _C * _L)], b16_b.at[p], psem.at[p])
            pltpu.async_copy(g_h.at[pl.ds(base * _L, _C * _L)], g16_b.at[p], psem.at[p])
            pltpu.async_copy(r1_h.at[pl.ds(base, _C)], r1_b.at[p], psem.at[p])
            if second:
                pltpu.async_copy(r2_h.at[pl.ds(base, _C)], r2_b.at[p], psem.at[p])
            pltpu.async_copy(p1_h.at[pl.ds(base, _C)], p1_b.at[p], psem.at[p])
            pltpu.async_copy(p2_h.at[pl.ds(base, _C)], p2_b.at[p], psem.at[p])

        def prep(p):
            pltpu.make_async_copy(a_h.at[pl.ds(0, _C * _L)], a16_b.at[p], psem.at[p]).wait()
            pltpu.make_async_copy(b_h.at[pl.ds(0, _C * _L)], b16_b.at[p], psem.at[p]).wait()
            pltpu.make_async_copy(g_h.at[pl.ds(0, _C * _L)], g16_b.at[p], psem.at[p]).wait()
            pltpu.make_async_copy(r1_h.at[pl.ds(0, _C)], r1_b.at[p], psem.at[p]).wait()
            if second:
                pltpu.make_async_copy(r2_h.at[pl.ds(0, _C)], r2_b.at[p], psem.at[p]).wait()
            pltpu.make_async_copy(p1_h.at[pl.ds(0, _C)], p1_b.at[p], psem.at[p]).wait()
            pltpu.make_async_copy(p2_h.at[pl.ds(0, _C)], p2_b.at[p], psem.at[p]).wait()
            for t in range(_C // _L):
                r1a_b[p, pl.ds(t * _L, _L)] = r1_b[p, pl.ds(t * _L, _L)] + coff
                if second:
                    r2a_b[p, pl.ds(t * _L, _L)] = r2_b[p, pl.ds(t * _L, _L)] + coff
            pltpu.async_copy(yT_h.at[r1a_b.at[p]], y1_b.at[p], gsem.at[p])
            if second:
                pltpu.async_copy(yT_h.at[r2a_b.at[p]], y2_b.at[p], gsem.at[p])

        def wait_gathers(p):
            pltpu.make_async_copy(yT_h.at[r1a_b.at[p]], y1_b.at[p], gsem.at[p]).wait()
            if second:
                pltpu.make_async_copy(yT_h.at[r2a_b.at[p]], y2_b.at[p], gsem.at[p]).wait()

        def compute(p):
            a16 = a16_b.at[p]
            b16 = b16_b.at[p]
            g16 = g16_b.at[p]
            y1v = y1_b.at[p]
            y2v = y2_b.at[p]
            termv = term_b.at[p]
            ntermv = nterm_b.at[p]

            def jbody(j, inner):
                al = a16[pl.ds(j * _L, _L)]
                be = b16[pl.ds(j * _L, _L)]
                ga = g16[pl.ds(j * _L, _L)]
                for t in range(_NT):
                    logT, nTinv, den = meds[t]
                    rate = al * jnp.exp(be * logT + ga * nTinv)
                    if second:
                        rate = rate * den
                        term = rate * y1v[j, pl.ds(t * _L, _L)] * y2v[j, pl.ds(t * _L, _L)]
                    else:
                        term = rate * y1v[j, pl.ds(t * _L, _L)]
                    termv[j, pl.ds(t * _L, _L)] = term
                    ntermv[j, pl.ds(t * _L, _L)] = -term
                return inner

            lax.fori_loop(0, _C, jbody, 0)

        def issue_scatters(p):
            for t in range(_C // _L):
                rs1_b[p, pl.ds(t * _L, _L)] = r1_b[p, pl.ds(t * _L, _L)]
                if second:
                    rs2_b[p, pl.ds(t * _L, _L)] = r2_b[p, pl.ds(t * _L, _L)]
                sp1_b[p, pl.ds(t * _L, _L)] = p1_b[p, pl.ds(t * _L, _L)]
                sp2_b[p, pl.ds(t * _L, _L)] = p2_b[p, pl.ds(t * _L, _L)]
            pltpu.async_copy(nterm_b.at[p], dy_sp.at[rs1_b.at[p]], ssem.at[p], add=True)
            if second:
                pltpu.async_copy(nterm_b.at[p], dy_sp.at[rs2_b.at[p]], ssem.at[p], add=True)
            pltpu.async_copy(term_b.at[p], dy_sp.at[sp1_b.at[p]], ssem.at[p], add=True)
            pltpu.async_copy(term_b.at[p], dy_sp.at[sp2_b.at[p]], ssem.at[p], add=True)

        def wait_scatters(p):
            pltpu.make_async_copy(nterm_b.at[p], dy_sp.at[rs1_b.at[p]], ssem.at[p]).wait()
            if second:
                pltpu.make_async_copy(nterm_b.at[p], dy_sp.at[rs2_b.at[p]], ssem.at[p]).wait()
            pltpu.make_async_copy(term_b.at[p], dy_sp.at[sp1_b.at[p]], ssem.at[p]).wait()
            pltpu.make_async_copy(term_b.at[p], dy_sp.at[sp2_b.at[p]], ssem.at[p]).wait()

        issue_params(0, 0)
        prep(0)
        issue_params(1, 1)

        prep(1)
        wait_gathers(0)
        compute(0)
        issue_scatters(0)
        issue_params(2, 0)
        prep(0)
        wait_gathers(1)
        compute(1)
        issue_scatters(1)
        issue_params(3, 1)

        def body(cp, carry):
            k0 = 2 * cp
            prep(1)
            wait_gathers(0)
            wait_scatters(0)
            compute(0)
            issue_scatters(0)
            issue_params(k0 + 2, 0)
            prep(0)
            wait_gathers(1)
            wait_scatters(1)
            compute(1)
            issue_scatters(1)
            issue_params(k0 + 3, 1)
            return carry

        lax.fori_loop(1, n // 2 - 1, body, 0)

        prep(1)
        wait_gathers(0)
        wait_scatters(0)
        compute(0)
        issue_scatters(0)
        wait_gathers(1)
        wait_scatters(1)
        compute(1)
        issue_scatters(1)
        wait_scatters(0)
        wait_scatters(1)

    run_phase(False, _T1, a1_h, b1_h, g1_h, r11_h, None, p11_h, p21_h)
    run_phase(True, _T2, a2_h, b2_h, g2_h, r12_h, r22_h, p12_h, p22_h)

    plsc.subcore_barrier()
    pltpu.sync_copy(dy_sp.at[pl.ds(s * nrow, nrow)],
                    out_h.at[c, pl.ds(s * nrow, nrow)])


def _tc_body(t_ref, wT_ref, bT_ref, wd_ref, bd_ref, y_ref,
             al_ref, be_ref, ga_ref, r1_ref, r2_ref, p1_ref, p2_ref,
             out_ref, *, n_spec):
    i = pl.program_id(0)

    t = t_ref[...]
    T = jnp.exp(wT_ref[0] * t + bT_ref[0]) + 10.0
    den = jnp.exp(wd_ref[0] * t + bd_ref[0])
    logT = jnp.log(T / 300.0)
    nTinv = -1.0 / T

    al = al_ref[0]
    be = be_ref[0]
    ga = ga_ref[0]
    rates = den * al * jnp.exp(be * logT + ga * nTinv)

    r1 = r1_ref[0]
    r2 = r2_ref[0]
    p1 = p1_ref[0]
    p2 = p2_ref[0]

    spec = lax.broadcasted_iota(jnp.int32, (n_spec, _RB), 0)
    G1 = (spec == r1).astype(jnp.float32)
    G2 = (spec == r2).astype(jnp.float32)
    P1 = (spec == p1).astype(jnp.float32)
    P2 = (spec == p2).astype(jnp.float32)

    y = y_ref[...]
    Y1 = jnp.dot(y, G1, preferred_element_type=jnp.float32)
    Y2 = jnp.dot(y, G2, preferred_element_type=jnp.float32)

    term = rates * Y1 * Y2

    S = P1 + P2 - G1 - G2
    contrib = lax.dot_general(term, S, (((1,), (1,)), ((), ())),
                              preferred_element_type=jnp.float32)

    @pl.when(i == 0)
    def _():
        out_ref[...] = jnp.zeros_like(out_ref)

    out_ref[...] += contrib


def _bcast16(x, n_pad):
    x = jnp.pad(x.astype(jnp.float32), (0, n_pad - x.shape[0]))
    return jnp.repeat(x[:, None], _L, axis=1).reshape(-1)


def _padi(x, n_pad):
    return jnp.pad(x.astype(jnp.int32), (0, n_pad - x.shape[0]))


def kernel(t_in, y_in, alpha_1st, beta_1st, gamma_1st, alpha_2nd, beta_2nd,
           gamma_2nd, w_T, b_T, w_d, b_d, inds_r1_1st, inds_p1_1st,
           inds_p2_1st, inds_r1_2nd, inds_r2_2nd, inds_p1_2nd, inds_p2_2nd):
    B = t_in.shape[0]
    nspec = y_in.shape[1]
    f32 = jnp.float32
    i32 = jnp.int32
    n1p = _T1 * _NTL
    n2p = _T2 * _NTL

    T_gas = jnp.exp(w_T * t_in + b_T) + 10.0
    den = jnp.exp(w_d * t_in + b_d).astype(f32)
    logT = jnp.log(T_gas / 300.0).astype(f32)
    nTinv = (-1.0 / T_gas).astype(f32)

    y = y_in.astype(f32)
    yT = jnp.concatenate([y[:_HB].T, y[_HB:].T], axis=0)

    scratch = [
        pltpu.VMEM((_HB,), f32),
        pltpu.VMEM((_HB,), f32),
        pltpu.VMEM((_HB,), f32),
        pltpu.VMEM((2, _C * _L), f32),
        pltpu.VMEM((2, _C * _L), f32),
        pltpu.VMEM((2, _C * _L), f32),
        pltpu.VMEM((2, _C), i32),
        pltpu.VMEM((2, _C), i32),
        pltpu.VMEM((2, _C), i32),
        pltpu.VMEM((2, _C), i32),
        pltpu.VMEM((2, _C), i32),
        pltpu.VMEM((2, _C), i32),
        pltpu.VMEM((2, _C), i32),
        pltpu.VMEM((2, _C), i32),
        pltpu.VMEM((2, _C), i32),
        pltpu.VMEM((2, _C), i32),
        pltpu.VMEM((2, _C, _HB), f32),
        pltpu.VMEM((2, _C, _HB), f32),
        pltpu.VMEM((2, _C, _HB), f32),
        pltpu.VMEM((2, _C, _HB), f32),
        pltpu.SemaphoreType.DMA((2,)),
        pltpu.SemaphoreType.DMA((2,)),
        pltpu.SemaphoreType.DMA((2,)),
        pltpu.VMEM_SHARED((nspec, _HB), f32),
    ]

    sc_call = pl.kernel(
        _sc_body,
        out_type=jax.ShapeDtypeStruct((_NSC, nspec, _HB), f32),
        mesh=plsc.VectorSubcoreMesh(core_axis_name="c", subcore_axis_name="s"),
        scratch_types=scratch,
    )
    sc_out = sc_call(
        logT, nTinv, den, yT,
        _bcast16(alpha_1st, n1p), _bcast16(beta_1st, n1p),
        _bcast16(gamma_1st, n1p),
        _padi(inds_r1_1st, n1p), _padi(inds_p1_1st, n1p),
        _padi(inds_p2_1st, n1p),
        _bcast16(alpha_2nd[:n2p], n2p), _bcast16(beta_2nd[:n2p], n2p),
        _bcast16(gamma_2nd[:n2p], n2p),
        _padi(inds_r1_2nd[:n2p], n2p), _padi(inds_r2_2nd[:n2p], n2p),
        _padi(inds_p1_2nd[:n2p], n2p), _padi(inds_p2_2nd[:n2p], n2p),
    )
    dy_sc = jnp.concatenate([sc_out[0], sc_out[1]], axis=1).T

    R2n = alpha_2nd.shape[0] - n2p
    nb = -(-R2n // _RB)
    pad = nb * _RB - R2n

    def padded(x, fill):
        x = jnp.pad(x[n2p:], (0, pad), constant_values=fill)
        return x.reshape(nb, 1, _RB)

    al = padded(alpha_2nd.astype(f32), 0.0)
    be = padded(beta_2nd.astype(f32), 0.0)
    ga = padded(gamma_2nd.astype(f32), 0.0)
    r1 = padded(inds_r1_2nd.astype(i32), 0)
    r2 = padded(inds_r2_2nd.astype(i32), 0)
    p1 = padded(inds_p1_2nd.astype(i32), 0)
    p2 = padded(inds_p2_2nd.astype(i32), 0)

    t2 = t_in.reshape(B, 1)

    full2d = lambda shape: pl.BlockSpec(shape, lambda i: (0, 0))
    par3d = pl.BlockSpec((1, 1, _RB), lambda i: (i, 0, 0))
    smem = pl.BlockSpec(memory_space=pltpu.SMEM)

    dy_tc = pl.pallas_call(
        functools.partial(_tc_body, n_spec=nspec),
        grid=(nb,),
        in_specs=[full2d((B, 1)), smem, smem, smem, smem, full2d((B, nspec)),
                  par3d, par3d, par3d, par3d, par3d, par3d, par3d],
        out_specs=pl.BlockSpec((B, nspec), lambda i: (0, 0)),
        out_shape=jax.ShapeDtypeStruct((B, nspec), f32),
    )(t2, w_T, b_T, w_d, b_d, y, al, be, ga, r1, r2, p1, p2)

    return dy_sc + dy_tc
